# Initial kernel scaffold; baseline (speedup 1.0000x reference)
import jax
import jax.numpy as jnp
from jax.experimental import pallas as pl


def kernel(h0, h1, h2, seg0, seg1, seg2, W1, b1, W2, b2):
    raise NotImplementedError("write your pallas kernel here")


# speedup vs baseline: 4.7230x; 4.7230x over previous
import functools

import jax
import jax.numpy as jnp
from jax import lax
from jax.experimental import pallas as pl
from jax.experimental.pallas import tpu as pltpu
from jax.experimental.pallas import tpu_sc as plsc

NG = 256
D = 128
NT = 3
R = 128
CW = 128


def _sc_segment_sums(n):
  info = plsc.get_sparse_core_info()
  nc, ns = info.num_cores, info.num_subcores
  nw = nc * ns
  nfull = n // R
  rem = n - nfull * R
  rows_per_tile = NG // ns

  mesh = plsc.VectorSubcoreMesh(core_axis_name="c", subcore_axis_name="s")

  out_type = [
      jax.ShapeDtypeStruct((nc, NT, NG, D), jnp.float32),
      jax.ShapeDtypeStruct((nc, NT, NG, CW), jnp.float32),
  ]
  scratch = [
      pltpu.VMEM((R,), jnp.int32),
      pltpu.VMEM((R, D), jnp.float32),
      pltpu.VMEM((R, CW), jnp.float32),
      pltpu.VMEM((rem, ), jnp.int32) if rem else pltpu.VMEM((8,), jnp.int32),
      pltpu.VMEM((max(rem, 1), D), jnp.float32),
  ] + [pltpu.VMEM_SHARED((NG, D), jnp.float32) for _ in range(NT)] \
    + [pltpu.VMEM_SHARED((NG, CW), jnp.float32) for _ in range(NT)]

  @functools.partial(pl.kernel, mesh=mesh, out_type=out_type,
                     scratch_types=scratch)
  def k(h0, s0, h1, s1, h2, s2, ones_hbm, zacc_hbm, zcnt_hbm,
        acc_out, cnt_out,
        seg_v, rows_v, ones_v, segr_v, rowsr_v,
        acc0_sh, acc1_sh, acc2_sh, cnt0_sh, cnt1_sh, cnt2_sh):
    accs = (acc0_sh, acc1_sh, acc2_sh)
    cnts = (cnt0_sh, cnt1_sh, cnt2_sh)
    c = lax.axis_index("c")
    s = lax.axis_index("s")
    w = s * nc + c

    zsl = pl.ds(s * rows_per_tile, rows_per_tile)
    for t in range(NT):
      pltpu.sync_copy(zacc_hbm.at[zsl], accs[t].at[zsl])
      pltpu.sync_copy(zcnt_hbm.at[zsl], cnts[t].at[zsl])
    pltpu.sync_copy(ones_hbm, ones_v)
    plsc.subcore_barrier()

    nk = (nfull - w + nw - 1) // nw
    for t, (h, seg) in enumerate(((h0, s0), (h1, s1), (h2, s2))):
      def body(i, carry, h=h, seg=seg, t=t):
        base = (w + i * nw) * R
        pltpu.sync_copy(seg.at[pl.ds(base, R)], seg_v)
        pltpu.sync_copy(h.at[pl.ds(base, R)], rows_v)
        pltpu.sync_copy(rows_v, accs[t].at[seg_v], add=True)
        pltpu.sync_copy(ones_v, cnts[t].at[seg_v], add=True)
        return carry
      lax.fori_loop(0, nk, body, 0)

    if rem:
      @pl.when(w == nw - 1)
      def _():
        for t, (h, seg) in enumerate(((h0, s0), (h1, s1), (h2, s2))):
          pltpu.sync_copy(seg.at[pl.ds(nfull * R, rem)], segr_v)
          pltpu.sync_copy(h.at[pl.ds(nfull * R, rem)], rowsr_v)
          pltpu.sync_copy(rowsr_v, accs[t].at[segr_v], add=True)
          pltpu.sync_copy(ones_v.at[pl.ds(0, rem)], cnts[t].at[segr_v],
                          add=True)

    plsc.subcore_barrier()

    for t in range(NT):
      pltpu.sync_copy(accs[t].at[zsl], acc_out.at[c, t, zsl])
      pltpu.sync_copy(cnts[t].at[zsl], cnt_out.at[c, t, zsl])

  return k


def _attention_tc(acc, cnt, W1, b1, W2):
  def body(acc_ref, cnt_ref, W1_ref, b1_ref, W2_ref, out_ref):
    w1 = W1_ref[...]
    b1v = b1_ref[...]
    w2 = W2_ref[...]
    zs, ss = [], []
    for t in range(NT):
      a = acc_ref[0, t] + acc_ref[1, t]
      cT = cnt_ref[0, t, :, 0:1] + cnt_ref[1, t, :, 0:1]
      z = a / jnp.maximum(cT, 1.0)
      zs.append(z)
      hzs = jnp.tanh(jnp.dot(z, w1, preferred_element_type=jnp.float32)
                     + b1v[None, :])
      ss.append(jnp.dot(hzs, w2, preferred_element_type=jnp.float32))
    sstack = jnp.concatenate(ss, axis=1)
    m = jnp.max(sstack, axis=1, keepdims=True)
    e = jnp.exp(sstack - m)
    beta = e / jnp.sum(e, axis=1, keepdims=True)
    out = beta[:, 0:1] * zs[0] + beta[:, 1:2] * zs[1] + beta[:, 2:3] * zs[2]
    out_ref[...] = out

  return pl.pallas_call(
      body,
      out_shape=jax.ShapeDtypeStruct((NG, D), jnp.float32),
  )(acc, cnt, W1, b1, W2)


def kernel(h0, h1, h2, seg0, seg1, seg2, W1, b1, W2, b2):
  n = h0.shape[0]
  ones = jnp.ones((R, CW), jnp.float32)
  zacc = jnp.zeros((NG, D), jnp.float32)
  zcnt = jnp.zeros((NG, CW), jnp.float32)
  sc = _sc_segment_sums(n)
  acc, cnt = sc(h0, seg0.astype(jnp.int32), h1, seg1.astype(jnp.int32),
                h2, seg2.astype(jnp.int32), ones, zacc, zcnt)
  return _attention_tc(acc, cnt, W1, b1, W2)


# speedup vs baseline: 7.0919x; 1.5016x over previous
import functools

import jax
import jax.numpy as jnp
from jax import lax
from jax.experimental import pallas as pl
from jax.experimental.pallas import tpu as pltpu
from jax.experimental.pallas import tpu_sc as plsc

NG = 256
D = 128
NT = 3
R = 128
CW = 128


def _sc_segment_sums(n):
  info = plsc.get_sparse_core_info()
  nc, ns = info.num_cores, info.num_subcores
  nw = nc * ns
  nfull = n // R
  rem = n - nfull * R
  rows_per_tile = NG // ns

  mesh = plsc.VectorSubcoreMesh(core_axis_name="c", subcore_axis_name="s")

  out_type = [
      jax.ShapeDtypeStruct((nc, NT, NG, D), jnp.float32),
      jax.ShapeDtypeStruct((nc, NT, NG, CW), jnp.float32),
  ]
  scratch = [
      pltpu.VMEM((R,), jnp.int32),
      pltpu.VMEM((R,), jnp.int32),
      pltpu.VMEM((R, D), jnp.float32),
      pltpu.VMEM((R, D), jnp.float32),
      pltpu.VMEM((R, CW), jnp.float32),
      pltpu.VMEM((rem, ), jnp.int32) if rem else pltpu.VMEM((8,), jnp.int32),
      pltpu.VMEM((max(rem, 1), D), jnp.float32),
      pltpu.SemaphoreType.DMA,
      pltpu.SemaphoreType.DMA,
  ] + [pltpu.VMEM_SHARED((NG, D), jnp.float32) for _ in range(NT)] \
    + [pltpu.VMEM_SHARED((NG, CW), jnp.float32) for _ in range(NT)]

  @functools.partial(pl.kernel, mesh=mesh, out_type=out_type,
                     scratch_types=scratch)
  def k(h0, s0, h1, s1, h2, s2, ones_hbm, zacc_hbm, zcnt_hbm,
        acc_out, cnt_out,
        seg_v0, seg_v1, rows_v0, rows_v1, ones_v, segr_v, rowsr_v,
        sem0, sem1,
        acc0_sh, acc1_sh, acc2_sh, cnt0_sh, cnt1_sh, cnt2_sh):
    accs = (acc0_sh, acc1_sh, acc2_sh)
    cnts = (cnt0_sh, cnt1_sh, cnt2_sh)
    bufs = ((seg_v0, rows_v0, sem0), (seg_v1, rows_v1, sem1))
    c = lax.axis_index("c")
    s = lax.axis_index("s")
    w = s * nc + c

    zsl = pl.ds(s * rows_per_tile, rows_per_tile)
    for t in range(NT):
      pltpu.sync_copy(zacc_hbm.at[zsl], accs[t].at[zsl])
      pltpu.sync_copy(zcnt_hbm.at[zsl], cnts[t].at[zsl])
    pltpu.sync_copy(ones_hbm, ones_v)
    plsc.subcore_barrier()

    nk = (nfull - w + nw - 1) // nw

    def issue(i, segb, rowsb, sem, seg, h):
      base = (w + i * nw) * R
      pltpu.async_copy(seg.at[pl.ds(base, R)], segb, sem)
      pltpu.async_copy(h.at[pl.ds(base, R)], rowsb, sem)

    def drain(segb, rowsb, sem, seg, h):
      pltpu.make_async_copy(seg.at[pl.ds(0, R)], segb, sem).wait()
      pltpu.make_async_copy(h.at[pl.ds(0, R)], rowsb, sem).wait()

    for t, (h, seg) in enumerate(((h0, s0), (h1, s1), (h2, s2))):
      issue(0, *bufs[0], seg, h)

      def body(i, carry, h=h, seg=seg, t=t):
        for p in range(2):
          @pl.when(lax.rem(i, 2) == p)
          def _(p=p):
            segb, rowsb, sem = bufs[p]
            drain(segb, rowsb, sem, seg, h)
            @pl.when(i + 1 < nk)
            def _():
              issue(i + 1, *bufs[1 - p], seg, h)
            pltpu.sync_copy(rowsb, accs[t].at[segb], add=True)
            pltpu.sync_copy(ones_v, cnts[t].at[segb], add=True)
        return carry
      lax.fori_loop(0, nk, body, 0)

    if rem:
      @pl.when(w == nw - 1)
      def _():
        for t, (h, seg) in enumerate(((h0, s0), (h1, s1), (h2, s2))):
          pltpu.sync_copy(seg.at[pl.ds(nfull * R, rem)], segr_v)
          pltpu.sync_copy(h.at[pl.ds(nfull * R, rem)], rowsr_v)
          pltpu.sync_copy(rowsr_v, accs[t].at[segr_v], add=True)
          pltpu.sync_copy(ones_v.at[pl.ds(0, rem)], cnts[t].at[segr_v],
                          add=True)

    plsc.subcore_barrier()

    for t in range(NT):
      pltpu.sync_copy(accs[t].at[zsl], acc_out.at[c, t, zsl])
      pltpu.sync_copy(cnts[t].at[zsl], cnt_out.at[c, t, zsl])

  return k


def _attention_tc(acc, cnt, W1, b1, W2):
  def body(acc_ref, cnt_ref, W1_ref, b1_ref, W2_ref, out_ref):
    w1 = W1_ref[...]
    b1v = b1_ref[...]
    w2 = W2_ref[...]
    zs, ss = [], []
    for t in range(NT):
      a = acc_ref[0, t] + acc_ref[1, t]
      cT = cnt_ref[0, t, :, 0:1] + cnt_ref[1, t, :, 0:1]
      z = a / jnp.maximum(cT, 1.0)
      zs.append(z)
      hzs = jnp.tanh(jnp.dot(z, w1, preferred_element_type=jnp.float32)
                     + b1v[None, :])
      ss.append(jnp.dot(hzs, w2, preferred_element_type=jnp.float32))
    sstack = jnp.concatenate(ss, axis=1)
    m = jnp.max(sstack, axis=1, keepdims=True)
    e = jnp.exp(sstack - m)
    beta = e / jnp.sum(e, axis=1, keepdims=True)
    out = beta[:, 0:1] * zs[0] + beta[:, 1:2] * zs[1] + beta[:, 2:3] * zs[2]
    out_ref[...] = out

  return pl.pallas_call(
      body,
      out_shape=jax.ShapeDtypeStruct((NG, D), jnp.float32),
  )(acc, cnt, W1, b1, W2)


def kernel(h0, h1, h2, seg0, seg1, seg2, W1, b1, W2, b2):
  n = h0.shape[0]
  ones = jnp.ones((R, CW), jnp.float32)
  zacc = jnp.zeros((NG, D), jnp.float32)
  zcnt = jnp.zeros((NG, CW), jnp.float32)
  sc = _sc_segment_sums(n)
  acc, cnt = sc(h0, seg0.astype(jnp.int32), h1, seg1.astype(jnp.int32),
                h2, seg2.astype(jnp.int32), ones, zacc, zcnt)
  return _attention_tc(acc, cnt, W1, b1, W2)


# speedup vs baseline: 8.6161x; 1.2149x over previous
import functools

import jax
import jax.numpy as jnp
from jax import lax
from jax.experimental import pallas as pl
from jax.experimental.pallas import tpu as pltpu
from jax.experimental.pallas import tpu_sc as plsc

NG = 256
D = 128
NT = 3
R = 128
CW = 128


def _sc_segment_sums(n):
  info = plsc.get_sparse_core_info()
  nc, ns = info.num_cores, info.num_subcores
  nw = nc * ns
  nfull = n // R
  rem = n - nfull * R
  rows_per_tile = NG // ns

  mesh = plsc.VectorSubcoreMesh(core_axis_name="c", subcore_axis_name="s")

  out_type = [
      jax.ShapeDtypeStruct((nc, NT, NG, D), jnp.float32),
      jax.ShapeDtypeStruct((nc, NT, NG, CW), jnp.float32),
  ]
  scratch = [
      pltpu.VMEM((R,), jnp.int32),
      pltpu.VMEM((R,), jnp.int32),
      pltpu.VMEM((R, D), jnp.float32),
      pltpu.VMEM((R, D), jnp.float32),
      pltpu.VMEM((R, CW), jnp.float32),
      pltpu.VMEM((rem, ), jnp.int32) if rem else pltpu.VMEM((8,), jnp.int32),
      pltpu.VMEM((max(rem, 1), D), jnp.float32),
      pltpu.SemaphoreType.DMA,
      pltpu.SemaphoreType.DMA,
      pltpu.VMEM((16, D), jnp.float32),
      pltpu.VMEM((16, CW), jnp.float32),
  ] + [pltpu.VMEM_SHARED((NG, D), jnp.float32) for _ in range(NT)] \
    + [pltpu.VMEM_SHARED((NG, CW), jnp.float32) for _ in range(NT)]

  @functools.partial(pl.kernel, mesh=mesh, out_type=out_type,
                     scratch_types=scratch)
  def k(h0, s0, h1, s1, h2, s2, ones_hbm, zacc_hbm, zcnt_hbm,
        acc_out, cnt_out,
        seg_v0, seg_v1, rows_v0, rows_v1, ones_v, segr_v, rowsr_v,
        sem0, sem1, sums_v, cntr_v,
        acc0_sh, acc1_sh, acc2_sh, cnt0_sh, cnt1_sh, cnt2_sh):
    accs = (acc0_sh, acc1_sh, acc2_sh)
    cnts = (cnt0_sh, cnt1_sh, cnt2_sh)
    bufs = ((seg_v0, rows_v0, sem0), (seg_v1, rows_v1, sem1))
    c = lax.axis_index("c")
    s = lax.axis_index("s")
    w = s * nc + c

    zsl = pl.ds(s * rows_per_tile, rows_per_tile)
    for t in range(NT):
      pltpu.sync_copy(zacc_hbm.at[zsl], accs[t].at[zsl])
      pltpu.sync_copy(zcnt_hbm.at[zsl], cnts[t].at[zsl])
    pltpu.sync_copy(ones_hbm, ones_v)
    pltpu.sync_copy(zacc_hbm.at[pl.ds(0, 16)], sums_v)
    pltpu.sync_copy(zcnt_hbm.at[pl.ds(0, 16)], cntr_v)
    for j in range(CW // 16):
      cntr_v[0, pl.ds(16 * j, 16)] = jnp.full((16,), float(R), jnp.float32)
    plsc.subcore_barrier()

    nk = (nfull - w + nw - 1) // nw

    def issue(i, segb, rowsb, sem, seg, h):
      base = (w + i * nw) * R
      pltpu.async_copy(seg.at[pl.ds(base, R)], segb, sem)
      pltpu.async_copy(h.at[pl.ds(base, R)], rowsb, sem)

    def drain(segb, rowsb, sem, seg, h):
      pltpu.make_async_copy(seg.at[pl.ds(0, R)], segb, sem).wait()
      pltpu.make_async_copy(h.at[pl.ds(0, R)], rowsb, sem).wait()

    for t, (h, seg) in enumerate(((h0, s0), (h1, s1), (h2, s2))):
      issue(0, *bufs[0], seg, h)

      def body(i, carry, h=h, seg=seg, t=t):
        for p in range(2):
          @pl.when(lax.rem(i, 2) == p)
          def _(p=p):
            segb, rowsb, sem = bufs[p]
            drain(segb, rowsb, sem, seg, h)
            @pl.when(i + 1 < nk)
            def _():
              issue(i + 1, *bufs[1 - p], seg, h)
            v0 = segb[pl.ds(0, 16)]
            vlast = segb[pl.ds(R - 16, 16)]
            uni = v0[0] == vlast[15]

            @pl.when(uni)
            def _():
              def sbody(r, acc):
                out = []
                for j in range(D // 16):
                  a = acc[j]
                  for u in range(8):
                    a = a + rowsb[8 * r + u, pl.ds(16 * j, 16)]
                  out.append(a)
                return tuple(out)
              acc = lax.fori_loop(
                  0, R // 8, sbody,
                  tuple(jnp.zeros((16,), jnp.float32)
                        for _ in range(D // 16)))
              for j in range(D // 16):
                sums_v[0, pl.ds(16 * j, 16)] = acc[j]
              pltpu.sync_copy(sums_v, accs[t].at[v0], add=True)
              pltpu.sync_copy(cntr_v, cnts[t].at[v0], add=True)

            @pl.when(jnp.logical_not(uni))
            def _():
              pltpu.sync_copy(rowsb, accs[t].at[segb], add=True)
              pltpu.sync_copy(ones_v, cnts[t].at[segb], add=True)
        return carry
      lax.fori_loop(0, nk, body, 0)

    if rem:
      @pl.when(w == nw - 1)
      def _():
        for t, (h, seg) in enumerate(((h0, s0), (h1, s1), (h2, s2))):
          pltpu.sync_copy(seg.at[pl.ds(nfull * R, rem)], segr_v)
          pltpu.sync_copy(h.at[pl.ds(nfull * R, rem)], rowsr_v)
          pltpu.sync_copy(rowsr_v, accs[t].at[segr_v], add=True)
          pltpu.sync_copy(ones_v.at[pl.ds(0, rem)], cnts[t].at[segr_v],
                          add=True)

    plsc.subcore_barrier()

    for t in range(NT):
      pltpu.sync_copy(accs[t].at[zsl], acc_out.at[c, t, zsl])
      pltpu.sync_copy(cnts[t].at[zsl], cnt_out.at[c, t, zsl])

  return k


def _attention_tc(acc, cnt, W1, b1, W2):
  def body(acc_ref, cnt_ref, W1_ref, b1_ref, W2_ref, out_ref):
    w1 = W1_ref[...]
    b1v = b1_ref[...]
    w2 = W2_ref[...]
    zs, ss = [], []
    for t in range(NT):
      a = acc_ref[0, t] + acc_ref[1, t]
      cT = cnt_ref[0, t, :, 0:1] + cnt_ref[1, t, :, 0:1]
      z = a / jnp.maximum(cT, 1.0)
      zs.append(z)
      hzs = jnp.tanh(jnp.dot(z, w1, preferred_element_type=jnp.float32)
                     + b1v[None, :])
      ss.append(jnp.dot(hzs, w2, preferred_element_type=jnp.float32))
    sstack = jnp.concatenate(ss, axis=1)
    m = jnp.max(sstack, axis=1, keepdims=True)
    e = jnp.exp(sstack - m)
    beta = e / jnp.sum(e, axis=1, keepdims=True)
    out = beta[:, 0:1] * zs[0] + beta[:, 1:2] * zs[1] + beta[:, 2:3] * zs[2]
    out_ref[...] = out

  return pl.pallas_call(
      body,
      out_shape=jax.ShapeDtypeStruct((NG, D), jnp.float32),
  )(acc, cnt, W1, b1, W2)


def kernel(h0, h1, h2, seg0, seg1, seg2, W1, b1, W2, b2):
  n = h0.shape[0]
  ones = jnp.ones((R, CW), jnp.float32)
  zacc = jnp.zeros((NG, D), jnp.float32)
  zcnt = jnp.zeros((NG, CW), jnp.float32)
  sc = _sc_segment_sums(n)
  acc, cnt = sc(h0, seg0.astype(jnp.int32), h1, seg1.astype(jnp.int32),
                h2, seg2.astype(jnp.int32), ones, zacc, zcnt)
  return _attention_tc(acc, cnt, W1, b1, W2)
